# baseline (speedup 1.0000x reference)
import functools

import jax
import jax.numpy as jnp
from jax import lax
from jax.experimental import pallas as pl
from jax.experimental.pallas import tpu as pltpu
from jax.experimental.pallas import tpu_sc as plsc

_NC = 2
_NS = 16
_NW = _NC * _NS
_BB = 128
_NBUF = 4
_L = 16


@functools.partial(jax.jit, static_argnums=(2, 3, 4))
def _gather(table, idxt, h_dim, b_dim, d):
    n = h_dim * b_dim
    units = n // _BB
    u_per_w = units // _NW
    b_per_w = u_per_w * _BB
    dt_dim = d // 8
    bt_dim = b_dim // _BB

    mesh = plsc.VectorSubcoreMesh(core_axis_name="c", subcore_axis_name="s")

    @functools.partial(
        pl.kernel,
        out_type=jax.ShapeDtypeStruct((h_dim, dt_dim, bt_dim, 8, 128),
                                      jnp.float32),
        mesh=mesh,
        compiler_params=pltpu.CompilerParams(
            use_tc_tiling_on_sc=False, needs_layout_passes=False
        ),
        scratch_types=[
            pltpu.VMEM((b_per_w,), jnp.int32),
            [pltpu.VMEM((_BB, d), jnp.float32) for _ in range(_NBUF)],
            [pltpu.VMEM((d, _BB), jnp.float32) for _ in range(_NBUF)],
            [pltpu.SemaphoreType.DMA for _ in range(_NBUF)],
            [pltpu.SemaphoreType.DMA for _ in range(_NBUF)],
        ],
    )
    def k(table_hbm, idx_hbm, out_hbm, idx_v, rows, sbufs, gsems, ssems):
        wid = lax.axis_index("s") * _NC + lax.axis_index("c")
        pltpu.sync_copy(idx_hbm.at[pl.ds(wid * b_per_w, b_per_w)], idx_v)
        u_base = wid * u_per_w
        base_ids = [(lax.iota(jnp.int32, _L) + j * _L) * d
                    for j in range(_BB // _L)]

        def fire(t, p):
            pltpu.async_copy(
                table_hbm.at[idx_v.at[pl.ds(t * _BB, _BB)]],
                rows[p], gsems[p],
            )

        def wait_gather(p):
            pltpu.make_async_copy(
                table_hbm.at[idx_v.at[pl.ds(0, _BB)]], rows[p], gsems[p]
            ).wait()

        row_ids = [lax.iota(jnp.int32, _L) + j * _L for j in range(_BB // _L)]

        def transpose(p):
            def body(dd, carry):
                col = jnp.full((_L,), dd, jnp.int32)
                for j in range(_BB // _L):
                    vec = plsc.load_gather(rows[p], [row_ids[j], col])
                    sbufs[p][dd, pl.ds(j * _L, _L)] = vec
                return carry
            lax.fori_loop(0, d, body, 0)

        def stores(t, p):
            u = u_base + t
            hh = u // bt_dim
            bt = lax.rem(u, bt_dim)
            for dt in range(dt_dim):
                pltpu.async_copy(
                    sbufs[p].at[pl.ds(dt * 8, 8)],
                    out_hbm.at[hh, dt, bt], ssems[p],
                )

        def wait_stores(p):
            for dt in range(dt_dim):
                pltpu.make_async_copy(
                    sbufs[p].at[pl.ds(dt * 8, 8)], out_hbm.at[0, 0, 0],
                    ssems[p],
                ).wait()

        for p in range(_NBUF):
            fire(p, p)

        def ring(i, carry):
            for p in range(_NBUF):
                t = _NBUF * i + p
                wait_gather(p)
                transpose(p)
                fire(t + _NBUF, p)
                stores(t, p)
                wait_stores(p)
            return carry

        lax.fori_loop(0, u_per_w // _NBUF - 1, ring, 0)

        for p in range(_NBUF):
            t = u_per_w - _NBUF + p
            wait_gather(p)
            transpose(p)
            stores(t, p)
            wait_stores(p)

    return k(table, idxt)


def kernel(inputs, embeddings):
    b, h = inputs.shape
    d = embeddings.shape[-1]
    idxt = inputs.T.reshape(-1).astype(jnp.int32)
    out5 = _gather(embeddings, idxt, h, b, d)
    return jnp.transpose(out5, (2, 4, 0, 1, 3)).reshape(b, h, d)


# speedup vs baseline: 1.5102x; 1.5102x over previous
import functools

import jax
import jax.numpy as jnp
from jax import lax
from jax.experimental import pallas as pl
from jax.experimental.pallas import tpu as pltpu
from jax.experimental.pallas import tpu_sc as plsc

_NC = 2
_NS = 16
_NW = _NC * _NS
_CHUNK = 256
_NBUF = 5


@functools.partial(jax.jit, static_argnums=(2, 3))
def _gather(table, idx, n, d):
    b_per_w = n // _NW
    chunks = b_per_w // _CHUNK

    mesh = plsc.VectorSubcoreMesh(core_axis_name="c", subcore_axis_name="s")

    @functools.partial(
        pl.kernel,
        out_type=jax.ShapeDtypeStruct((n, d), jnp.float32),
        mesh=mesh,
        compiler_params=pltpu.CompilerParams(use_tc_tiling_on_sc=False),
        scratch_types=[
            pltpu.VMEM((b_per_w,), jnp.int32),
            [pltpu.VMEM((_CHUNK, d), jnp.float32) for _ in range(_NBUF)],
            [pltpu.SemaphoreType.DMA for _ in range(_NBUF)],
            [pltpu.SemaphoreType.DMA for _ in range(_NBUF)],
        ],
    )
    def k(table_hbm, idx_hbm, out_hbm, idx_v, rows, gsems, ssems):
        wid = lax.axis_index("s") * _NC + lax.axis_index("c")
        pltpu.sync_copy(idx_hbm.at[pl.ds(wid * b_per_w, b_per_w)], idx_v)
        out_base = wid * b_per_w

        def fire(c, b):
            pltpu.async_copy(
                table_hbm.at[idx_v.at[pl.ds(c * _CHUNK, _CHUNK)]],
                rows[b],
                gsems[b],
            )

        def wait_gathers(b):
            pltpu.make_async_copy(
                table_hbm.at[idx_v.at[pl.ds(0, _CHUNK)]], rows[b], gsems[b]
            ).wait()

        def store(c, b):
            pltpu.async_copy(
                rows[b], out_hbm.at[pl.ds(out_base + c * _CHUNK, _CHUNK)],
                ssems[b],
            )

        def wait_store(b):
            pltpu.make_async_copy(
                rows[b], out_hbm.at[pl.ds(out_base, _CHUNK)], ssems[b]
            ).wait()

        for b in range(_NBUF):
            fire(b, b)

        def ring(i, carry):
            for b in range(_NBUF):
                c = _NBUF * i + b
                wait_gathers(b)
                store(c, b)
                wait_store(b)
                fire(c + _NBUF, b)
            return carry

        lax.fori_loop(0, chunks // _NBUF - 1, ring, 0)

        for b in range(_NBUF):
            wait_gathers(b)
            store(chunks - _NBUF + b, b)
            wait_store(b)

    return k(table, idx)


def kernel(inputs, embeddings):
    b, h = inputs.shape
    d = embeddings.shape[-1]
    flat = inputs.reshape(-1).astype(jnp.int32)
    out = _gather(embeddings, flat, flat.shape[0], d)
    return out.reshape(b, h, d)
